# baseline (speedup 1.0000x reference)
import functools
import jax
import jax.numpy as jnp
from jax import lax

_PH = jax.lax.Precision.HIGHEST
from jax.experimental import pallas as pl
from jax.experimental.pallas import tpu as pltpu
from jax.experimental.pallas import tpu_sc as plsc

N = 10000
NP = 10112
E = 160000
HID = 128
NB = 16
NL = 3
NC = 10
EPS = 1e-5

NCORE = 2
NSUB = 16
NW = NCORE * NSUB
EW = E // NW
CH = 40
NCHUNK = EW // CH
NROWS = NP // NSUB

_mesh = lambda: plsc.VectorSubcoreMesh(core_axis_name="c", subcore_axis_name="s")


def _bdot(a, b):
    return jnp.dot(a.astype(jnp.bfloat16), b.astype(jnp.bfloat16),
                   preferred_element_type=jnp.float32)


def _b16(a):
    return a.astype(jnp.bfloat16).astype(jnp.float32)


CH2 = 64
NFULL = EW // CH2
TAIL = EW - NFULL * CH2


@functools.partial(jax.jit, static_argnames=("width", "ew"))
def _sc_gather(rtab, ctab, row2, col2, *, width, ew=EW):
    @functools.partial(
        pl.kernel,
        mesh=_mesh(),
        out_type=[
            jax.ShapeDtypeStruct((NW * ew, width), jnp.float32),
            jax.ShapeDtypeStruct((NW * ew, width), jnp.float32),
        ],
        scratch_types=[
            pltpu.VMEM((ew,), jnp.int32),
            pltpu.VMEM((ew,), jnp.int32),
            pltpu.VMEM((CH2, width), jnp.float32),
            pltpu.VMEM((CH2, width), jnp.float32),
            pltpu.VMEM((CH2, width), jnp.float32),
            pltpu.VMEM((CH2, width), jnp.float32),
            pltpu.VMEM((CH2, width), jnp.float32),
            pltpu.VMEM((CH2, width), jnp.float32),
            pltpu.SemaphoreType.DMA,
            pltpu.SemaphoreType.DMA,
            pltpu.SemaphoreType.DMA,
            pltpu.SemaphoreType.DMA,
            pltpu.SemaphoreType.DMA,
            pltpu.SemaphoreType.DMA,
        ],
    )
    def gk(rt, ct, r2, c2, rg_out, cg_out, ridx, cidx,
           rb0, rb1, rb2, cb0, cb1, cb2, sr0, sr1, sr2, sc0, sc1, sc2):
        wid = lax.axis_index("s") * NCORE + lax.axis_index("c")
        base = pl.multiple_of(wid * ew, 8)
        pltpu.sync_copy(r2.at[wid], ridx)
        pltpu.sync_copy(c2.at[wid], cidx)
        rb = (rb0, rb1, rb2)
        cb = (cb0, cb1, cb2)
        sr = (sr0, sr1, sr2)
        sc = (sc0, sc1, sc2)
        nfull = ew // CH2
        tail = ew - nfull * CH2
        nch = nfull + (1 if tail else 0)
        pend = {}

        def drain(k):
            hr, hc, sz = pend.pop(k)
            hr.wait()
            hc.wait()
            sl = k % 3
            out = pl.ds(base + k * CH2, sz)
            pltpu.sync_copy(rb[sl].at[pl.ds(0, sz)], rg_out.at[out])
            pltpu.sync_copy(cb[sl].at[pl.ds(0, sz)], cg_out.at[out])

        for k in range(nch):
            sz = CH2 if k < nfull else tail
            sl = k % 3
            isl = pl.ds(k * CH2, sz)
            hr = pltpu.async_copy(rt.at[ridx.at[isl]],
                                  rb[sl].at[pl.ds(0, sz)], sr[sl])
            hc = pltpu.async_copy(ct.at[cidx.at[isl]],
                                  cb[sl].at[pl.ds(0, sz)], sc[sl])
            pend[k] = (hr, hc, sz)
            if k >= 2:
                drain(k - 2)
        drain(nch - 2)
        drain(nch - 1)

    return gk(rtab, ctab, row2, col2)


@jax.jit
def _sc_gather1(tab, idx3):
    @functools.partial(
        pl.kernel,
        mesh=_mesh(),
        out_type=jax.ShapeDtypeStruct((E, HID), jnp.float32),
        scratch_types=[
            pltpu.VMEM((NCHUNK, CH), jnp.int32),
            pltpu.VMEM((CH, HID), jnp.float32),
            pltpu.SemaphoreType.DMA,
        ],
    )
    def gk(tb, i3, out, idx, buf, sem):
        wid = lax.axis_index("s") * NCORE + lax.axis_index("c")
        base = wid * EW
        pltpu.sync_copy(i3.at[wid], idx)

        def body(k, carry):
            pltpu.async_copy(tb.at[idx.at[k]], buf, sem).wait()
            off = pl.multiple_of(base + k * CH, 8)
            pltpu.sync_copy(buf, out.at[pl.ds(off, CH)])
            return carry

        lax.fori_loop(0, NCHUNK, body, 0)

    return gk(tab, idx3)


_SB = 1280
_NBLK = E // _SB
_NT = NP // HID
_G = _NBLK + 2 * _NT


@functools.partial(jax.jit, static_argnames=("nblk",))
def _tc_segsum(eoS, dS, colS3, b_of_g, t_of_g, f_of_g, *, nblk=_NBLK):
    def body(b_ref, t_ref, f_ref, colr, eor, dr, acc_ref, ds_ref):
        g = pl.program_id(0)
        t = t_ref[g]
        f = f_ref[g]
        colv = colr[0]
        loc = colv - t * HID
        rows = lax.broadcasted_iota(jnp.int32, (HID, 1), 0)
        oh = jnp.where(rows == loc, 1.0, 0.0)
        part = jnp.dot(oh, eor[...], precision=_PH)
        partd = jnp.dot(oh, dr[...], precision=_PH)

        @pl.when(f == 3)
        def _():
            acc_ref[...] = part
            ds_ref[...] = partd

        @pl.when(f == 1)
        def _():
            acc_ref[...] = acc_ref[...] + part
            ds_ref[...] = ds_ref[...] + partd

    grid_spec = pltpu.PrefetchScalarGridSpec(
        num_scalar_prefetch=3,
        grid=(nblk + 2 * _NT,),
        in_specs=[
            pl.BlockSpec((1, 1, _SB), lambda g, b, t, f: (b[g], 0, 0)),
            pl.BlockSpec((_SB, HID), lambda g, b, t, f: (b[g], 0)),
            pl.BlockSpec((_SB, 16), lambda g, b, t, f: (b[g], 0)),
        ],
        out_specs=[
            pl.BlockSpec((HID, HID), lambda g, b, t, f: (t[g], 0)),
            pl.BlockSpec((HID, 16), lambda g, b, t, f: (t[g], 0)),
        ],
    )
    return pl.pallas_call(
        body,
        grid_spec=grid_spec,
        out_shape=[
            jax.ShapeDtypeStruct((NP + HID, HID), jnp.float32),
            jax.ShapeDtypeStruct((NP + HID, 16), jnp.float32),
        ],
    )(b_of_g, t_of_g, f_of_g, colS3, eoS, dS)


def _segsum_schedule(colS, nblk=_NBLK):
    G = nblk + 2 * _NT
    bfirst = colS[::_SB] // HID
    blast = colS[_SB - 1::_SB] // HID
    t_ids = jnp.arange(_NT)
    bstart = jnp.searchsorted(blast, t_ids, side="left")
    bend = jnp.searchsorted(bfirst, t_ids, side="right")
    n_t = jnp.maximum(bend - bstart, 1)
    off = jnp.concatenate([jnp.zeros((1,), jnp.int32),
                           jnp.cumsum(n_t).astype(jnp.int32)])
    total = off[-1]
    g_ids = jnp.arange(G, dtype=jnp.int32)
    t_of_g = jnp.searchsorted(off, g_ids, side="right").astype(jnp.int32) - 1
    t_of_g = jnp.clip(t_of_g, 0, _NT - 1)
    j = g_ids - off[t_of_g]
    empty = (bend - bstart)[t_of_g] <= 0
    b_of_g = jnp.where(empty, 0, bstart[t_of_g] + j).astype(jnp.int32)
    f_of_g = jnp.where(j == 0, 3, 1).astype(jnp.int32)
    pad = g_ids >= total
    t_of_g = jnp.where(pad, _NT, t_of_g).astype(jnp.int32)
    b_of_g = jnp.where(pad, 0, b_of_g)
    f_of_g = jnp.where(pad, 3, f_of_g)
    return b_of_g, t_of_g, f_of_g


_EB = 2000


@functools.partial(jax.jit, static_argnames=("hw", "ne"))
def _tc_edge(rg, cg, eah, wd, wa, be, wg, bg, wc, bc, *, hw, ne=E):
    W = hw + 128

    def body(rgr, cgr, eahr, wdr, war, ber, wgr, bgr, wcr, bcr, eo_ref, d_ref):
        rgv = rgr[...]
        cgv = cgr[...]
        ev = cgv[:, hw:hw + 16] - rgv[:, hw:hw + 16]
        s = jnp.sum(ev * ev, axis=1, keepdims=True)
        dist = jnp.where(s > 0, jnp.sqrt(jnp.where(s > 0, s, 1.0)), 0.0)
        t = _b16(dist) * _b16(wdr[...]) + _bdot(eahr[...], war[...]) + ber[...]
        if hw:
            t = t + rgv[:, :hw] + cgv[:, :hw]
        e1 = jax.nn.relu(t)
        g = jax.nn.sigmoid(jnp.sum(_b16(e1) * _b16(wgr[...]), axis=1,
                                   keepdims=True) + bgr[...][:, :1])
        eo = e1 * g
        w = (jnp.sum(_b16(eo) * _b16(wcr[...]), axis=1, keepdims=True)
             + bcr[...][:, :1])
        lane3 = (lax.broadcasted_iota(jnp.int32, (1, 16), 1) == 3)
        d_ref[...] = ev * w + jnp.where(lane3, 1.0, 0.0)
        eo_ref[...] = eo

    grid = (ne // _EB,)
    full = lambda shape: pl.BlockSpec(shape, lambda i: (0, 0))
    return pl.pallas_call(
        body,
        grid=grid,
        in_specs=[
            pl.BlockSpec((_EB, W), lambda i: (i, 0)),
            pl.BlockSpec((_EB, W), lambda i: (i, 0)),
            pl.BlockSpec((_EB, NB), lambda i: (i, 0)),
            full((1, HID)), full((NB, HID)), full((1, HID)),
            full((1, HID)), full((1, HID)), full((1, HID)), full((1, HID)),
        ],
        out_specs=[
            pl.BlockSpec((_EB, HID), lambda i: (i, 0)),
            pl.BlockSpec((_EB, 16), lambda i: (i, 0)),
        ],
        out_shape=[
            jax.ShapeDtypeStruct((ne, HID), jnp.float32),
            jax.ShapeDtypeStruct((ne, 16), jnp.float32),
        ],
    )(rg, cg, eah, wd, wa, be, wg, bg, wc, bc)


_NBK = 1264


@functools.partial(jax.jit, static_argnames=("has_h", "do_prep"))
def _tc_node(acc, accB, dsm, dsmB, cv, dv, xcur, hn_in, hres, wna_h, wna_a,
             bna, wnb, bnb, ln_g, ln_b, wrow, wcol, *, has_h, do_prep):
    def body(*refs):
        i = 0
        accr = refs[i]; i += 1
        accbr = refs[i]; i += 1
        dsr = refs[i]; i += 1
        dsbr = refs[i]; i += 1
        cvr = refs[i]; i += 1
        dvr = refs[i]; i += 1
        xr = refs[i]; i += 1
        if has_h:
            hnr = refs[i]; i += 1
            hrr = refs[i]; i += 1
            wnahr = refs[i]; i += 1
        wnaar = refs[i]; i += 1
        bnar = refs[i]; i += 1
        wnbr = refs[i]; i += 1
        bnbr = refs[i]; i += 1
        if do_prep:
            lngr = refs[i]; i += 1
            lnbr = refs[i]; i += 1
            wrr = refs[i]; i += 1
            wcr = refs[i]; i += 1
        h_out = refs[i]; i += 1
        x_out = refs[i]; i += 1
        if do_prep:
            r_out = refs[i]; i += 1
            c_out = refs[i]; i += 1
            hn_out = refs[i]; i += 1

        na = (accr[...] + accbr[...]) / cvr[...]
        t = _bdot(na, wnaar[...]) + bnar[...]
        if has_h:
            t = t + _bdot(hnr[...], wnahr[...])
        node_out = jax.nn.relu(_bdot(jax.nn.relu(t), wnbr[...]) + bnbr[...])
        h_new = node_out + hrr[...] if has_h else node_out
        lane = lax.broadcasted_iota(jnp.int32, (1, 16), 1)
        x_new = (xr[...]
                 + jnp.where(lane < 3, dsr[...] + dsbr[...], 0.0) / dvr[...])
        h_out[...] = h_new
        x_out[...] = x_new
        if do_prep:
            m = jnp.mean(h_new, axis=1, keepdims=True)
            v = jnp.mean((h_new - m) * (h_new - m), axis=1, keepdims=True)
            hn2 = (h_new - m) / jnp.sqrt(v + EPS) * lngr[...] + lnbr[...]
            zpad = jnp.zeros((x_new.shape[0], HID - 16), jnp.float32)
            r_out[...] = jnp.concatenate([_bdot(hn2, wrr[...]), x_new, zpad], axis=1)
            c_out[...] = jnp.concatenate([_bdot(hn2, wcr[...]), x_new, zpad], axis=1)
            hn_out[...] = hn2

    grid = (NP // _NBK,)
    full = lambda shape: pl.BlockSpec(shape, lambda i: tuple(0 for _ in shape))
    in_specs = [
        pl.BlockSpec((_NBK, HID), lambda i: (i, 0)),
        pl.BlockSpec((_NBK, HID), lambda i: (i, 0)),
        pl.BlockSpec((_NBK, 16), lambda i: (i, 0)),
        pl.BlockSpec((_NBK, 16), lambda i: (i, 0)),
        pl.BlockSpec((_NBK, 1), lambda i: (i, 0)),
        pl.BlockSpec((_NBK, 1), lambda i: (i, 0)),
        pl.BlockSpec((_NBK, 16), lambda i: (i, 0)),
    ]
    args = [acc, accB, dsm, dsmB, cv, dv, xcur]
    if has_h:
        in_specs += [pl.BlockSpec((_NBK, HID), lambda i: (i, 0))] * 2
        in_specs += [full((HID, HID))]
        args += [hn_in, hres, wna_h]
    in_specs += [full((HID, HID)), full((1, HID)), full((HID, HID)),
                 full((1, HID))]
    args += [wna_a, bna, wnb, bnb]
    if do_prep:
        in_specs += [full((1, HID)), full((1, HID)), full((HID, HID)),
                     full((HID, HID))]
        args += [ln_g, ln_b, wrow, wcol]
    out_specs = [
        pl.BlockSpec((_NBK, HID), lambda i: (i, 0)),
        pl.BlockSpec((_NBK, 16), lambda i: (i, 0)),
    ]
    out_shape = [
        jax.ShapeDtypeStruct((NP, HID), jnp.float32),
        jax.ShapeDtypeStruct((NP, 16), jnp.float32),
    ]
    if do_prep:
        out_specs += [pl.BlockSpec((_NBK, 2 * HID), lambda i: (i, 0))] * 2
        out_shape += [jax.ShapeDtypeStruct((NP, 2 * HID), jnp.float32)] * 2
        out_specs += [pl.BlockSpec((_NBK, HID), lambda i: (i, 0))]
        out_shape += [jax.ShapeDtypeStruct((NP, HID), jnp.float32)]
    return pl.pallas_call(
        body, grid=grid, in_specs=in_specs, out_specs=out_specs,
        out_shape=out_shape)(*args)


@jax.jit
def _tc_head(h, fn_g, fn_b, w1, b1, w2, b2, w3p, b3p):
    def body(hr, gr, br, w1r, b1r, w2r, b2r, w3r, b3r, log_ref, emb_ref):
        hv = hr[...]
        m = jnp.mean(hv, axis=1, keepdims=True)
        v = jnp.mean((hv - m) * (hv - m), axis=1, keepdims=True)
        emb = (hv - m) / jnp.sqrt(v + EPS) * gr[...] + br[...]
        z = jax.nn.relu(_bdot(emb, w1r[...]) + b1r[...])
        z = jax.nn.relu(_bdot(z, w2r[...]) + b2r[...])
        log_ref[...] = _bdot(z, w3r[...]) + b3r[...]
        emb_ref[...] = emb

    grid = (NP // _NBK,)
    full = lambda shape: pl.BlockSpec(shape, lambda i: (0, 0))
    return pl.pallas_call(
        body, grid=grid,
        in_specs=[pl.BlockSpec((_NBK, HID), lambda i: (i, 0)),
                  full((1, HID)), full((1, HID)), full((HID, HID)),
                  full((1, HID)), full((HID, HID)), full((1, HID)),
                  full((HID, HID)), full((1, HID))],
        out_specs=[pl.BlockSpec((_NBK, HID), lambda i: (i, 0)),
                   pl.BlockSpec((_NBK, HID), lambda i: (i, 0))],
        out_shape=[jax.ShapeDtypeStruct((NP, HID), jnp.float32),
                   jax.ShapeDtypeStruct((NP, HID), jnp.float32)],
    )(h, fn_g, fn_b, w1, b1, w2, b2, w3p, b3p)


@jax.jit
def _tc_prep(hn_in, ln_g, ln_b, wrow, wcol, x):
    def body(hr, gr, br, wrr, wcr, xr, r_out, c_out, hn_out):
        hv = hr[...]
        m = jnp.mean(hv, axis=1, keepdims=True)
        v = jnp.mean((hv - m) * (hv - m), axis=1, keepdims=True)
        hn2 = (hv - m) / jnp.sqrt(v + EPS) * gr[...] + br[...]
        xv = xr[...]
        zpad = jnp.zeros((xv.shape[0], HID - 16), jnp.float32)
        r_out[...] = jnp.concatenate([_bdot(hn2, wrr[...]), xv, zpad], axis=1)
        c_out[...] = jnp.concatenate([_bdot(hn2, wcr[...]), xv, zpad], axis=1)
        hn_out[...] = hn2

    grid = (NP // _NBK,)
    full = lambda shape: pl.BlockSpec(shape, lambda i: (0, 0))
    return pl.pallas_call(
        body, grid=grid,
        in_specs=[pl.BlockSpec((_NBK, HID), lambda i: (i, 0)),
                  full((1, HID)), full((1, HID)), full((HID, HID)),
                  full((HID, HID)),
                  pl.BlockSpec((_NBK, 16), lambda i: (i, 0))],
        out_specs=[pl.BlockSpec((_NBK, 2 * HID), lambda i: (i, 0)),
                   pl.BlockSpec((_NBK, 2 * HID), lambda i: (i, 0)),
                   pl.BlockSpec((_NBK, HID), lambda i: (i, 0))],
        out_shape=[jax.ShapeDtypeStruct((NP, 2 * HID), jnp.float32),
                   jax.ShapeDtypeStruct((NP, 2 * HID), jnp.float32),
                   jax.ShapeDtypeStruct((NP, HID), jnp.float32)],
    )(hn_in, ln_g, ln_b, wrow, wcol, x)


def kernel(pos_norm, edge_index, edge_angle_hist, center_mask,
           We0, be0, Wc0, bc0, Wg0, bg0, Wna0, bna0, Wnb0, bnb0,
           We, be, Wc, bc, Wg, bg, Wna, bna, Wnb, bnb,
           ln_g, ln_b, fn_g, fn_b, Wc1, bc1, Wc2, bc2, Wc3, bc3):
    f32 = jnp.float32
    x = jnp.pad(pos_norm, ((0, NP - N), (0, 13)))
    row, col = edge_index[0], edge_index[1]
    eah = edge_angle_hist

    perm = jnp.argsort(col)
    rowP = row[perm]
    colP = col[perm]
    perm3 = perm.astype(jnp.int32).reshape(NW, NCHUNK, CH)
    EA = 64000
    EB = E - EA
    NBA = EA // _SB
    NBB = EB // _SB
    rowA2 = rowP[:EA].reshape(NW, EA // NW)
    colA2 = colP[:EA].reshape(NW, EA // NW)
    rowB2 = rowP[EA:].reshape(NW, EB // NW)
    colB2 = colP[EA:].reshape(NW, EB // NW)
    colS3A = colP[:EA].reshape(NBA, 1, _SB)
    colS3B = colP[EA:].reshape(NBB, 1, _SB)
    schedA = _segsum_schedule(colP[:EA], NBA)
    schedB = _segsum_schedule(colP[EA:], NBB)
    cv = jnp.clip(jnp.bincount(col, length=NP).astype(f32), 1.0, None)[:, None]
    dv = jnp.clip(jnp.bincount(row, length=NP).astype(f32), 1.0, None)[:, None]

    eah128 = jnp.pad(eah, ((0, 0), (0, HID - NB)))
    eahP = _sc_gather1(eah128, perm3)[:, :NB]
    eahPA = eahP[:EA]
    eahPB = eahP[EA:]

    r2 = lambda a: a.reshape(1, -1)

    x128 = jnp.pad(x, ((0, 0), (0, HID - 16)))

    def edge_pass(rt_, ct_, w, wd, wa, beL, wgL, bgL, wcL, bcL, hw):
        rgA, cgA = _sc_gather(rt_, ct_, rowA2, colA2, width=w, ew=EA // NW)
        rgB, cgB = _sc_gather(rt_, ct_, rowB2, colB2, width=w, ew=EB // NW)
        eoA, dA = _tc_edge(rgA, cgA, eahPA, wd, wa, beL, wgL, bgL, wcL, bcL,
                           hw=hw, ne=EA)
        accA, dsA = _tc_segsum(eoA, dA, colS3A, *schedA, nblk=NBA)
        eoB, dB = _tc_edge(rgB, cgB, eahPB, wd, wa, beL, wgL, bgL, wcL, bcL,
                           hw=hw, ne=EB)
        accB, dsB = _tc_segsum(eoB, dB, colS3B, *schedB, nblk=NBB)
        return accA[:NP], accB[:NP], dsA[:NP], dsB[:NP]

    aA, aB, dA_, dB_ = edge_pass(
        x128, x128, HID, r2(We0[0]), We0[1:], r2(be0),
        r2(Wg0[:, 0]), jnp.broadcast_to(bg0, (1, HID)),
        r2(Wc0[:, 0]), jnp.broadcast_to(bc0, (1, HID)), 0)
    h, x = _tc_node(aA, aB, dA_, dB_, cv, dv, x, None, None, None,
                    Wna0, r2(bna0), Wnb0, r2(bnb0), None, None, None, None,
                    has_h=False, do_prep=False)
    rt, ct, hn = _tc_prep(h, r2(ln_g[0]), r2(ln_b[0]), We[0][:HID],
                          We[0][HID:2 * HID], x)

    for i in range(NL):
        Wei = We[i]
        aA, aB, dA_, dB_ = edge_pass(
            rt, ct, 2 * HID, r2(Wei[2 * HID]), Wei[2 * HID + 1:],
            r2(be[i]), r2(Wg[i][:, 0]),
            jnp.broadcast_to(bg[i], (1, HID)),
            r2(Wc[i][:, 0]), jnp.broadcast_to(bc[i], (1, HID)), HID)
        do_prep = i + 1 < NL
        outs = _tc_node(
            aA, aB, dA_, dB_, cv, dv, x, hn, h, Wna[i][:HID],
            Wna[i][HID:], r2(bna[i]), Wnb[i], r2(bnb[i]),
            r2(ln_g[i + 1]) if do_prep else None,
            r2(ln_b[i + 1]) if do_prep else None,
            We[i + 1][:HID] if do_prep else None,
            We[i + 1][HID:2 * HID] if do_prep else None,
            has_h=True, do_prep=do_prep)
        if do_prep:
            h, x, rt, ct, hn = outs
        else:
            h, x = outs

    w3p = jnp.pad(Wc3, ((0, 0), (0, HID - NC)))
    b3p = jnp.pad(bc3, ((0, HID - NC)))
    logits_p, emb = _tc_head(h, r2(fn_g), r2(fn_b), Wc1, r2(bc1),
                             Wc2, r2(bc2), w3p, r2(b3p))
    return logits_p[:N, :NC], emb[:N]


# speedup vs baseline: 1.1668x; 1.1668x over previous
import functools
import jax
import jax.numpy as jnp
from jax import lax

_PH = jax.lax.Precision.HIGHEST
from jax.experimental import pallas as pl
from jax.experimental.pallas import tpu as pltpu
from jax.experimental.pallas import tpu_sc as plsc

N = 10000
NP = 10112
E = 160000
HID = 128
NB = 16
NL = 3
NC = 10
EPS = 1e-5

NCORE = 2
NSUB = 16
NW = NCORE * NSUB
EW = E // NW
CH = 40
NCHUNK = EW // CH
NROWS = NP // NSUB

_mesh = lambda: plsc.VectorSubcoreMesh(core_axis_name="c", subcore_axis_name="s")


def _bdot(a, b):
    return jnp.dot(a.astype(jnp.bfloat16), b.astype(jnp.bfloat16),
                   preferred_element_type=jnp.float32)


def _b16(a):
    return a.astype(jnp.bfloat16).astype(jnp.float32)


CH2 = 96
NFULL = EW // CH2
TAIL = EW - NFULL * CH2


@functools.partial(jax.jit, static_argnames=("width",))
def _sc_gather(rtab, ctab, row2, col2, *, width):
    @functools.partial(
        pl.kernel,
        mesh=_mesh(),
        out_type=[
            jax.ShapeDtypeStruct((E, width), jnp.float32),
            jax.ShapeDtypeStruct((E, width), jnp.float32),
        ],
        scratch_types=[
            pltpu.VMEM((EW,), jnp.int32),
            pltpu.VMEM((EW,), jnp.int32),
            pltpu.VMEM((CH2, width), jnp.float32),
            pltpu.VMEM((CH2, width), jnp.float32),
            pltpu.VMEM((CH2, width), jnp.float32),
            pltpu.VMEM((CH2, width), jnp.float32),
            pltpu.SemaphoreType.DMA,
            pltpu.SemaphoreType.DMA,
            pltpu.SemaphoreType.DMA,
            pltpu.SemaphoreType.DMA,
        ],
    )
    def gk(rt, ct, r2, c2, rg_out, cg_out, ridx, cidx,
           rb0, rb1, cb0, cb1, sr0, sr1, sc0, sc1):
        wid = lax.axis_index("s") * NCORE + lax.axis_index("c")
        base = pl.multiple_of(wid * EW, 8)
        pltpu.sync_copy(r2.at[wid], ridx)
        pltpu.sync_copy(c2.at[wid], cidx)
        rb = (rb0, rb1)
        cb = (cb0, cb1)
        sr = (sr0, sr1)
        sc = (sc0, sc1)
        nch = NFULL + 1
        pend = {}

        def drain(k):
            hr, hc, sz = pend.pop(k)
            hr.wait()
            hc.wait()
            sl = k % 2
            out = pl.ds(base + k * CH2, sz)
            pltpu.sync_copy(rb[sl].at[pl.ds(0, sz)], rg_out.at[out])
            pltpu.sync_copy(cb[sl].at[pl.ds(0, sz)], cg_out.at[out])

        for k in range(nch):
            sz = CH2 if k < NFULL else TAIL
            sl = k % 2
            isl = pl.ds(k * CH2, sz)
            hr = pltpu.async_copy(rt.at[ridx.at[isl]],
                                  rb[sl].at[pl.ds(0, sz)], sr[sl])
            hc = pltpu.async_copy(ct.at[cidx.at[isl]],
                                  cb[sl].at[pl.ds(0, sz)], sc[sl])
            pend[k] = (hr, hc, sz)
            if k >= 1:
                drain(k - 1)
        drain(nch - 1)

    return gk(rtab, ctab, row2, col2)


@jax.jit
def _sc_gather1(tab, idx3):
    @functools.partial(
        pl.kernel,
        mesh=_mesh(),
        out_type=jax.ShapeDtypeStruct((E, HID), jnp.float32),
        scratch_types=[
            pltpu.VMEM((NCHUNK, CH), jnp.int32),
            pltpu.VMEM((CH, HID), jnp.float32),
            pltpu.SemaphoreType.DMA,
        ],
    )
    def gk(tb, i3, out, idx, buf, sem):
        wid = lax.axis_index("s") * NCORE + lax.axis_index("c")
        base = wid * EW
        pltpu.sync_copy(i3.at[wid], idx)

        def body(k, carry):
            pltpu.async_copy(tb.at[idx.at[k]], buf, sem).wait()
            off = pl.multiple_of(base + k * CH, 8)
            pltpu.sync_copy(buf, out.at[pl.ds(off, CH)])
            return carry

        lax.fori_loop(0, NCHUNK, body, 0)

    return gk(tab, idx3)


_SB = 1280
_NBLK = E // _SB
_NT = NP // HID
_G = _NBLK + 2 * _NT


@jax.jit
def _tc_segsum(eoS, dS, colS3, b_of_g, t_of_g, f_of_g):
    def body(b_ref, t_ref, f_ref, colr, eor, dr, acc_ref, ds_ref):
        g = pl.program_id(0)
        t = t_ref[g]
        f = f_ref[g]
        colv = colr[0]
        loc = colv - t * HID
        rows = lax.broadcasted_iota(jnp.int32, (HID, 1), 0)
        oh = jnp.where(rows == loc, 1.0, 0.0)
        part = jnp.dot(oh, eor[...], precision=_PH)
        partd = jnp.dot(oh, dr[...], precision=_PH)

        @pl.when(f == 3)
        def _():
            acc_ref[...] = part
            ds_ref[...] = partd

        @pl.when(f == 1)
        def _():
            acc_ref[...] = acc_ref[...] + part
            ds_ref[...] = ds_ref[...] + partd

    grid_spec = pltpu.PrefetchScalarGridSpec(
        num_scalar_prefetch=3,
        grid=(_G,),
        in_specs=[
            pl.BlockSpec((1, 1, _SB), lambda g, b, t, f: (b[g], 0, 0)),
            pl.BlockSpec((_SB, HID), lambda g, b, t, f: (b[g], 0)),
            pl.BlockSpec((_SB, 16), lambda g, b, t, f: (b[g], 0)),
        ],
        out_specs=[
            pl.BlockSpec((HID, HID), lambda g, b, t, f: (t[g], 0)),
            pl.BlockSpec((HID, 16), lambda g, b, t, f: (t[g], 0)),
        ],
    )
    return pl.pallas_call(
        body,
        grid_spec=grid_spec,
        out_shape=[
            jax.ShapeDtypeStruct((NP + HID, HID), jnp.float32),
            jax.ShapeDtypeStruct((NP + HID, 16), jnp.float32),
        ],
    )(b_of_g, t_of_g, f_of_g, colS3, eoS, dS)


def _segsum_schedule(colS):
    bfirst = colS[::_SB] // HID
    blast = colS[_SB - 1::_SB] // HID
    t_ids = jnp.arange(_NT)
    bstart = jnp.searchsorted(blast, t_ids, side="left")
    bend = jnp.searchsorted(bfirst, t_ids, side="right")
    n_t = jnp.maximum(bend - bstart, 1)
    off = jnp.concatenate([jnp.zeros((1,), jnp.int32),
                           jnp.cumsum(n_t).astype(jnp.int32)])
    total = off[-1]
    g_ids = jnp.arange(_G, dtype=jnp.int32)
    t_of_g = jnp.searchsorted(off, g_ids, side="right").astype(jnp.int32) - 1
    t_of_g = jnp.clip(t_of_g, 0, _NT - 1)
    j = g_ids - off[t_of_g]
    empty = (bend - bstart)[t_of_g] <= 0
    b_of_g = jnp.where(empty, 0, bstart[t_of_g] + j).astype(jnp.int32)
    f_of_g = jnp.where(j == 0, 3, 1).astype(jnp.int32)
    pad = g_ids >= total
    t_of_g = jnp.where(pad, _NT, t_of_g).astype(jnp.int32)
    b_of_g = jnp.where(pad, 0, b_of_g)
    f_of_g = jnp.where(pad, 3, f_of_g)
    return b_of_g, t_of_g, f_of_g


_EB = 2000


@functools.partial(jax.jit, static_argnames=("hw",))
def _tc_edge(rg, cg, eah, wd, wa, be, wg, bg, wc, bc, *, hw):
    W = hw + 128

    def body(rgr, cgr, eahr, wdr, war, ber, wgr, bgr, wcr, bcr, eo_ref, d_ref):
        rgv = rgr[...]
        cgv = cgr[...]
        ev = cgv[:, hw:hw + 16] - rgv[:, hw:hw + 16]
        s = jnp.sum(ev * ev, axis=1, keepdims=True)
        dist = jnp.where(s > 0, jnp.sqrt(jnp.where(s > 0, s, 1.0)), 0.0)
        t = _b16(dist) * _b16(wdr[...]) + _bdot(eahr[...], war[...]) + ber[...]
        if hw:
            t = t + rgv[:, :hw] + cgv[:, :hw]
        e1 = jax.nn.relu(t)
        g = jax.nn.sigmoid(jnp.sum(_b16(e1) * _b16(wgr[...]), axis=1,
                                   keepdims=True) + bgr[...][:, :1])
        eo = e1 * g
        w = (jnp.sum(_b16(eo) * _b16(wcr[...]), axis=1, keepdims=True)
             + bcr[...][:, :1])
        lane3 = (lax.broadcasted_iota(jnp.int32, (1, 16), 1) == 3)
        d_ref[...] = ev * w + jnp.where(lane3, 1.0, 0.0)
        eo_ref[...] = eo

    grid = (E // _EB,)
    full = lambda shape: pl.BlockSpec(shape, lambda i: (0, 0))
    return pl.pallas_call(
        body,
        grid=grid,
        in_specs=[
            pl.BlockSpec((_EB, W), lambda i: (i, 0)),
            pl.BlockSpec((_EB, W), lambda i: (i, 0)),
            pl.BlockSpec((_EB, NB), lambda i: (i, 0)),
            full((1, HID)), full((NB, HID)), full((1, HID)),
            full((1, HID)), full((1, HID)), full((1, HID)), full((1, HID)),
        ],
        out_specs=[
            pl.BlockSpec((_EB, HID), lambda i: (i, 0)),
            pl.BlockSpec((_EB, 16), lambda i: (i, 0)),
        ],
        out_shape=[
            jax.ShapeDtypeStruct((E, HID), jnp.float32),
            jax.ShapeDtypeStruct((E, 16), jnp.float32),
        ],
    )(rg, cg, eah, wd, wa, be, wg, bg, wc, bc)


_NBK = 1264


@functools.partial(jax.jit, static_argnames=("has_h", "do_prep"))
def _tc_node(acc, dsm, cv, dv, xcur, hn_in, hres, wna_h, wna_a, bna, wnb, bnb,
             ln_g, ln_b, wrow, wcol, *, has_h, do_prep):
    def body(*refs):
        i = 0
        accr = refs[i]; i += 1
        dsr = refs[i]; i += 1
        cvr = refs[i]; i += 1
        dvr = refs[i]; i += 1
        xr = refs[i]; i += 1
        if has_h:
            hnr = refs[i]; i += 1
            hrr = refs[i]; i += 1
            wnahr = refs[i]; i += 1
        wnaar = refs[i]; i += 1
        bnar = refs[i]; i += 1
        wnbr = refs[i]; i += 1
        bnbr = refs[i]; i += 1
        if do_prep:
            lngr = refs[i]; i += 1
            lnbr = refs[i]; i += 1
            wrr = refs[i]; i += 1
            wcr = refs[i]; i += 1
        h_out = refs[i]; i += 1
        x_out = refs[i]; i += 1
        if do_prep:
            r_out = refs[i]; i += 1
            c_out = refs[i]; i += 1
            hn_out = refs[i]; i += 1

        na = accr[...] / cvr[...]
        t = _bdot(na, wnaar[...]) + bnar[...]
        if has_h:
            t = t + _bdot(hnr[...], wnahr[...])
        node_out = jax.nn.relu(_bdot(jax.nn.relu(t), wnbr[...]) + bnbr[...])
        h_new = node_out + hrr[...] if has_h else node_out
        lane = lax.broadcasted_iota(jnp.int32, (1, 16), 1)
        x_new = xr[...] + jnp.where(lane < 3, dsr[...], 0.0) / dvr[...]
        h_out[...] = h_new
        x_out[...] = x_new
        if do_prep:
            m = jnp.mean(h_new, axis=1, keepdims=True)
            v = jnp.mean((h_new - m) * (h_new - m), axis=1, keepdims=True)
            hn2 = (h_new - m) / jnp.sqrt(v + EPS) * lngr[...] + lnbr[...]
            zpad = jnp.zeros((x_new.shape[0], HID - 16), jnp.float32)
            r_out[...] = jnp.concatenate([_bdot(hn2, wrr[...]), x_new, zpad], axis=1)
            c_out[...] = jnp.concatenate([_bdot(hn2, wcr[...]), x_new, zpad], axis=1)
            hn_out[...] = hn2

    grid = (NP // _NBK,)
    full = lambda shape: pl.BlockSpec(shape, lambda i: tuple(0 for _ in shape))
    in_specs = [
        pl.BlockSpec((_NBK, HID), lambda i: (i, 0)),
        pl.BlockSpec((_NBK, 16), lambda i: (i, 0)),
        pl.BlockSpec((_NBK, 1), lambda i: (i, 0)),
        pl.BlockSpec((_NBK, 1), lambda i: (i, 0)),
        pl.BlockSpec((_NBK, 16), lambda i: (i, 0)),
    ]
    args = [acc, dsm, cv, dv, xcur]
    if has_h:
        in_specs += [pl.BlockSpec((_NBK, HID), lambda i: (i, 0))] * 2
        in_specs += [full((HID, HID))]
        args += [hn_in, hres, wna_h]
    in_specs += [full((HID, HID)), full((1, HID)), full((HID, HID)),
                 full((1, HID))]
    args += [wna_a, bna, wnb, bnb]
    if do_prep:
        in_specs += [full((1, HID)), full((1, HID)), full((HID, HID)),
                     full((HID, HID))]
        args += [ln_g, ln_b, wrow, wcol]
    out_specs = [
        pl.BlockSpec((_NBK, HID), lambda i: (i, 0)),
        pl.BlockSpec((_NBK, 16), lambda i: (i, 0)),
    ]
    out_shape = [
        jax.ShapeDtypeStruct((NP, HID), jnp.float32),
        jax.ShapeDtypeStruct((NP, 16), jnp.float32),
    ]
    if do_prep:
        out_specs += [pl.BlockSpec((_NBK, 2 * HID), lambda i: (i, 0))] * 2
        out_shape += [jax.ShapeDtypeStruct((NP, 2 * HID), jnp.float32)] * 2
        out_specs += [pl.BlockSpec((_NBK, HID), lambda i: (i, 0))]
        out_shape += [jax.ShapeDtypeStruct((NP, HID), jnp.float32)]
    return pl.pallas_call(
        body, grid=grid, in_specs=in_specs, out_specs=out_specs,
        out_shape=out_shape)(*args)


@jax.jit
def _tc_head(h, fn_g, fn_b, w1, b1, w2, b2, w3p, b3p):
    def body(hr, gr, br, w1r, b1r, w2r, b2r, w3r, b3r, log_ref, emb_ref):
        hv = hr[...]
        m = jnp.mean(hv, axis=1, keepdims=True)
        v = jnp.mean((hv - m) * (hv - m), axis=1, keepdims=True)
        emb = (hv - m) / jnp.sqrt(v + EPS) * gr[...] + br[...]
        z = jax.nn.relu(_bdot(emb, w1r[...]) + b1r[...])
        z = jax.nn.relu(_bdot(z, w2r[...]) + b2r[...])
        log_ref[...] = _bdot(z, w3r[...]) + b3r[...]
        emb_ref[...] = emb

    grid = (NP // _NBK,)
    full = lambda shape: pl.BlockSpec(shape, lambda i: (0, 0))
    return pl.pallas_call(
        body, grid=grid,
        in_specs=[pl.BlockSpec((_NBK, HID), lambda i: (i, 0)),
                  full((1, HID)), full((1, HID)), full((HID, HID)),
                  full((1, HID)), full((HID, HID)), full((1, HID)),
                  full((HID, HID)), full((1, HID))],
        out_specs=[pl.BlockSpec((_NBK, HID), lambda i: (i, 0)),
                   pl.BlockSpec((_NBK, HID), lambda i: (i, 0))],
        out_shape=[jax.ShapeDtypeStruct((NP, HID), jnp.float32),
                   jax.ShapeDtypeStruct((NP, HID), jnp.float32)],
    )(h, fn_g, fn_b, w1, b1, w2, b2, w3p, b3p)


@jax.jit
def _tc_prep(hn_in, ln_g, ln_b, wrow, wcol, x):
    def body(hr, gr, br, wrr, wcr, xr, r_out, c_out, hn_out):
        hv = hr[...]
        m = jnp.mean(hv, axis=1, keepdims=True)
        v = jnp.mean((hv - m) * (hv - m), axis=1, keepdims=True)
        hn2 = (hv - m) / jnp.sqrt(v + EPS) * gr[...] + br[...]
        xv = xr[...]
        zpad = jnp.zeros((xv.shape[0], HID - 16), jnp.float32)
        r_out[...] = jnp.concatenate([_bdot(hn2, wrr[...]), xv, zpad], axis=1)
        c_out[...] = jnp.concatenate([_bdot(hn2, wcr[...]), xv, zpad], axis=1)
        hn_out[...] = hn2

    grid = (NP // _NBK,)
    full = lambda shape: pl.BlockSpec(shape, lambda i: (0, 0))
    return pl.pallas_call(
        body, grid=grid,
        in_specs=[pl.BlockSpec((_NBK, HID), lambda i: (i, 0)),
                  full((1, HID)), full((1, HID)), full((HID, HID)),
                  full((HID, HID)),
                  pl.BlockSpec((_NBK, 16), lambda i: (i, 0))],
        out_specs=[pl.BlockSpec((_NBK, 2 * HID), lambda i: (i, 0)),
                   pl.BlockSpec((_NBK, 2 * HID), lambda i: (i, 0)),
                   pl.BlockSpec((_NBK, HID), lambda i: (i, 0))],
        out_shape=[jax.ShapeDtypeStruct((NP, 2 * HID), jnp.float32),
                   jax.ShapeDtypeStruct((NP, 2 * HID), jnp.float32),
                   jax.ShapeDtypeStruct((NP, HID), jnp.float32)],
    )(hn_in, ln_g, ln_b, wrow, wcol, x)


def kernel(pos_norm, edge_index, edge_angle_hist, center_mask,
           We0, be0, Wc0, bc0, Wg0, bg0, Wna0, bna0, Wnb0, bnb0,
           We, be, Wc, bc, Wg, bg, Wna, bna, Wnb, bnb,
           ln_g, ln_b, fn_g, fn_b, Wc1, bc1, Wc2, bc2, Wc3, bc3):
    f32 = jnp.float32
    x = jnp.pad(pos_norm, ((0, NP - N), (0, 13)))
    row, col = edge_index[0], edge_index[1]
    eah = edge_angle_hist

    perm = jnp.argsort(col)
    rowP = row[perm]
    colP = col[perm]
    perm3 = perm.astype(jnp.int32).reshape(NW, NCHUNK, CH)
    rowP2 = rowP.reshape(NW, EW)
    colP2 = colP.reshape(NW, EW)
    colS3 = colP.reshape(_NBLK, 1, _SB)
    b_of_g, t_of_g, f_of_g = _segsum_schedule(colP)
    cv = jnp.clip(jnp.bincount(col, length=NP).astype(f32), 1.0, None)[:, None]
    dv = jnp.clip(jnp.bincount(row, length=NP).astype(f32), 1.0, None)[:, None]

    eah128 = jnp.pad(eah, ((0, 0), (0, HID - NB)))
    eahP = _sc_gather1(eah128, perm3)[:, :NB]

    r2 = lambda a: a.reshape(1, -1)

    x128 = jnp.pad(x, ((0, 0), (0, HID - 16)))
    rg, cg = _sc_gather(x128, x128, rowP2, colP2, width=HID)
    eo, d = _tc_edge(rg, cg, eahP, r2(We0[0]), We0[1:], r2(be0),
                     r2(Wg0[:, 0]), jnp.broadcast_to(bg0, (1, HID)),
                     r2(Wc0[:, 0]), jnp.broadcast_to(bc0, (1, HID)), hw=0)
    acc_e, ds_e = _tc_segsum(eo, d, colS3, b_of_g, t_of_g, f_of_g)
    h, x = _tc_node(acc_e[:NP], ds_e[:NP], cv, dv, x, None, None, None,
                    Wna0, r2(bna0), Wnb0, r2(bnb0), None, None, None, None,
                    has_h=False, do_prep=False)
    rt, ct, hn = _tc_prep(h, r2(ln_g[0]), r2(ln_b[0]), We[0][:HID],
                          We[0][HID:2 * HID], x)

    for i in range(NL):
        Wei = We[i]
        rg, cg = _sc_gather(rt, ct, rowP2, colP2, width=2 * HID)
        eo, d = _tc_edge(rg, cg, eahP, r2(Wei[2 * HID]), Wei[2 * HID + 1:],
                         r2(be[i]), r2(Wg[i][:, 0]),
                         jnp.broadcast_to(bg[i], (1, HID)),
                         r2(Wc[i][:, 0]), jnp.broadcast_to(bc[i], (1, HID)),
                         hw=HID)
        acc_e, ds_e = _tc_segsum(eo, d, colS3, b_of_g, t_of_g, f_of_g)
        do_prep = i + 1 < NL
        outs = _tc_node(
            acc_e[:NP], ds_e[:NP], cv, dv, x, hn, h, Wna[i][:HID],
            Wna[i][HID:], r2(bna[i]), Wnb[i], r2(bnb[i]),
            r2(ln_g[i + 1]) if do_prep else None,
            r2(ln_b[i + 1]) if do_prep else None,
            We[i + 1][:HID] if do_prep else None,
            We[i + 1][HID:2 * HID] if do_prep else None,
            has_h=True, do_prep=do_prep)
        if do_prep:
            h, x, rt, ct, hn = outs
        else:
            h, x = outs

    w3p = jnp.pad(Wc3, ((0, 0), (0, HID - NC)))
    b3p = jnp.pad(bc3, ((0, HID - NC)))
    logits_p, emb = _tc_head(h, r2(fn_g), r2(fn_b), Wc1, r2(bc1),
                             Wc2, r2(bc2), w3p, r2(b3p))
    return logits_p[:N, :NC], emb[:N]


# speedup vs baseline: 1.1795x; 1.0108x over previous
import functools
import jax
import jax.numpy as jnp
from jax import lax

_PH = jax.lax.Precision.HIGHEST
from jax.experimental import pallas as pl
from jax.experimental.pallas import tpu as pltpu
from jax.experimental.pallas import tpu_sc as plsc

N = 10000
NP = 10112
E = 160000
HID = 128
NB = 16
NL = 3
NC = 10
EPS = 1e-5

NCORE = 2
NSUB = 16
NW = NCORE * NSUB
EW = E // NW
CH = 40
NCHUNK = EW // CH
NROWS = NP // NSUB

_mesh = lambda: plsc.VectorSubcoreMesh(core_axis_name="c", subcore_axis_name="s")


def _bdot(a, b):
    return jnp.dot(a.astype(jnp.bfloat16), b.astype(jnp.bfloat16),
                   preferred_element_type=jnp.float32)


def _b16(a):
    return a.astype(jnp.bfloat16).astype(jnp.float32)


CH2 = 64
NFULL = EW // CH2
TAIL = EW - NFULL * CH2


@functools.partial(jax.jit, static_argnames=("width",))
def _sc_gather(rtab, ctab, row2, col2, *, width):
    @functools.partial(
        pl.kernel,
        mesh=_mesh(),
        out_type=[
            jax.ShapeDtypeStruct((E, width), jnp.float32),
            jax.ShapeDtypeStruct((E, width), jnp.float32),
        ],
        scratch_types=[
            pltpu.VMEM((EW,), jnp.int32),
            pltpu.VMEM((EW,), jnp.int32),
            pltpu.VMEM((CH2, width), jnp.float32),
            pltpu.VMEM((CH2, width), jnp.float32),
            pltpu.VMEM((CH2, width), jnp.float32),
            pltpu.VMEM((CH2, width), jnp.float32),
            pltpu.VMEM((CH2, width), jnp.float32),
            pltpu.VMEM((CH2, width), jnp.float32),
            pltpu.SemaphoreType.DMA,
            pltpu.SemaphoreType.DMA,
            pltpu.SemaphoreType.DMA,
            pltpu.SemaphoreType.DMA,
            pltpu.SemaphoreType.DMA,
            pltpu.SemaphoreType.DMA,
        ],
    )
    def gk(rt, ct, r2, c2, rg_out, cg_out, ridx, cidx,
           rb0, rb1, rb2, cb0, cb1, cb2, sr0, sr1, sr2, sc0, sc1, sc2):
        wid = lax.axis_index("s") * NCORE + lax.axis_index("c")
        base = pl.multiple_of(wid * EW, 8)
        pltpu.sync_copy(r2.at[wid], ridx)
        pltpu.sync_copy(c2.at[wid], cidx)
        rb = (rb0, rb1, rb2)
        cb = (cb0, cb1, cb2)
        sr = (sr0, sr1, sr2)
        sc = (sc0, sc1, sc2)
        nch = NFULL + 1
        pend = {}

        def drain(k):
            hr, hc, sz = pend.pop(k)
            hr.wait()
            hc.wait()
            sl = k % 3
            out = pl.ds(base + k * CH2, sz)
            pltpu.sync_copy(rb[sl].at[pl.ds(0, sz)], rg_out.at[out])
            pltpu.sync_copy(cb[sl].at[pl.ds(0, sz)], cg_out.at[out])

        for k in range(nch):
            sz = CH2 if k < NFULL else TAIL
            sl = k % 3
            isl = pl.ds(k * CH2, sz)
            hr = pltpu.async_copy(rt.at[ridx.at[isl]],
                                  rb[sl].at[pl.ds(0, sz)], sr[sl])
            hc = pltpu.async_copy(ct.at[cidx.at[isl]],
                                  cb[sl].at[pl.ds(0, sz)], sc[sl])
            pend[k] = (hr, hc, sz)
            if k >= 2:
                drain(k - 2)
        drain(nch - 2)
        drain(nch - 1)

    return gk(rtab, ctab, row2, col2)


@jax.jit
def _sc_gather1(tab, idx3):
    @functools.partial(
        pl.kernel,
        mesh=_mesh(),
        out_type=jax.ShapeDtypeStruct((E, HID), jnp.float32),
        scratch_types=[
            pltpu.VMEM((NCHUNK, CH), jnp.int32),
            pltpu.VMEM((CH, HID), jnp.float32),
            pltpu.SemaphoreType.DMA,
        ],
    )
    def gk(tb, i3, out, idx, buf, sem):
        wid = lax.axis_index("s") * NCORE + lax.axis_index("c")
        base = wid * EW
        pltpu.sync_copy(i3.at[wid], idx)

        def body(k, carry):
            pltpu.async_copy(tb.at[idx.at[k]], buf, sem).wait()
            off = pl.multiple_of(base + k * CH, 8)
            pltpu.sync_copy(buf, out.at[pl.ds(off, CH)])
            return carry

        lax.fori_loop(0, NCHUNK, body, 0)

    return gk(tab, idx3)


_SB = 1280
_NBLK = E // _SB
_NT = NP // HID
_G = _NBLK + 2 * _NT


@jax.jit
def _tc_segsum(eoS, dS, colS3, b_of_g, t_of_g, f_of_g):
    def body(b_ref, t_ref, f_ref, colr, eor, dr, acc_ref, ds_ref):
        g = pl.program_id(0)
        t = t_ref[g]
        f = f_ref[g]
        colv = colr[0]
        loc = colv - t * HID
        rows = lax.broadcasted_iota(jnp.int32, (HID, 1), 0)
        oh = jnp.where(rows == loc, 1.0, 0.0)
        part = jnp.dot(oh, eor[...], precision=_PH)
        partd = jnp.dot(oh, dr[...], precision=_PH)

        @pl.when(f == 3)
        def _():
            acc_ref[...] = part
            ds_ref[...] = partd

        @pl.when(f == 1)
        def _():
            acc_ref[...] = acc_ref[...] + part
            ds_ref[...] = ds_ref[...] + partd

    grid_spec = pltpu.PrefetchScalarGridSpec(
        num_scalar_prefetch=3,
        grid=(_G,),
        in_specs=[
            pl.BlockSpec((1, 1, _SB), lambda g, b, t, f: (b[g], 0, 0)),
            pl.BlockSpec((_SB, HID), lambda g, b, t, f: (b[g], 0)),
            pl.BlockSpec((_SB, 16), lambda g, b, t, f: (b[g], 0)),
        ],
        out_specs=[
            pl.BlockSpec((HID, HID), lambda g, b, t, f: (t[g], 0)),
            pl.BlockSpec((HID, 16), lambda g, b, t, f: (t[g], 0)),
        ],
    )
    return pl.pallas_call(
        body,
        grid_spec=grid_spec,
        out_shape=[
            jax.ShapeDtypeStruct((NP + HID, HID), jnp.float32),
            jax.ShapeDtypeStruct((NP + HID, 16), jnp.float32),
        ],
    )(b_of_g, t_of_g, f_of_g, colS3, eoS, dS)


def _segsum_schedule(colS):
    bfirst = colS[::_SB] // HID
    blast = colS[_SB - 1::_SB] // HID
    t_ids = jnp.arange(_NT)
    bstart = jnp.searchsorted(blast, t_ids, side="left")
    bend = jnp.searchsorted(bfirst, t_ids, side="right")
    n_t = jnp.maximum(bend - bstart, 1)
    off = jnp.concatenate([jnp.zeros((1,), jnp.int32),
                           jnp.cumsum(n_t).astype(jnp.int32)])
    total = off[-1]
    g_ids = jnp.arange(_G, dtype=jnp.int32)
    t_of_g = jnp.searchsorted(off, g_ids, side="right").astype(jnp.int32) - 1
    t_of_g = jnp.clip(t_of_g, 0, _NT - 1)
    j = g_ids - off[t_of_g]
    empty = (bend - bstart)[t_of_g] <= 0
    b_of_g = jnp.where(empty, 0, bstart[t_of_g] + j).astype(jnp.int32)
    f_of_g = jnp.where(j == 0, 3, 1).astype(jnp.int32)
    pad = g_ids >= total
    t_of_g = jnp.where(pad, _NT, t_of_g).astype(jnp.int32)
    b_of_g = jnp.where(pad, 0, b_of_g)
    f_of_g = jnp.where(pad, 3, f_of_g)
    return b_of_g, t_of_g, f_of_g


_EB = 2000


@functools.partial(jax.jit, static_argnames=("hw",))
def _tc_edge(rg, cg, eah, wd, wa, be, wg, bg, wc, bc, *, hw):
    W = hw + 128

    def body(rgr, cgr, eahr, wdr, war, ber, wgr, bgr, wcr, bcr, eo_ref, d_ref):
        rgv = rgr[...]
        cgv = cgr[...]
        ev = cgv[:, hw:hw + 16] - rgv[:, hw:hw + 16]
        s = jnp.sum(ev * ev, axis=1, keepdims=True)
        dist = jnp.where(s > 0, jnp.sqrt(jnp.where(s > 0, s, 1.0)), 0.0)
        t = _b16(dist) * _b16(wdr[...]) + _bdot(eahr[...], war[...]) + ber[...]
        if hw:
            t = t + rgv[:, :hw] + cgv[:, :hw]
        e1 = jax.nn.relu(t)
        g = jax.nn.sigmoid(jnp.sum(_b16(e1) * _b16(wgr[...]), axis=1,
                                   keepdims=True) + bgr[...][:, :1])
        eo = e1 * g
        w = (jnp.sum(_b16(eo) * _b16(wcr[...]), axis=1, keepdims=True)
             + bcr[...][:, :1])
        lane3 = (lax.broadcasted_iota(jnp.int32, (1, 16), 1) == 3)
        d_ref[...] = ev * w + jnp.where(lane3, 1.0, 0.0)
        eo_ref[...] = eo

    grid = (E // _EB,)
    full = lambda shape: pl.BlockSpec(shape, lambda i: (0, 0))
    return pl.pallas_call(
        body,
        grid=grid,
        in_specs=[
            pl.BlockSpec((_EB, W), lambda i: (i, 0)),
            pl.BlockSpec((_EB, W), lambda i: (i, 0)),
            pl.BlockSpec((_EB, NB), lambda i: (i, 0)),
            full((1, HID)), full((NB, HID)), full((1, HID)),
            full((1, HID)), full((1, HID)), full((1, HID)), full((1, HID)),
        ],
        out_specs=[
            pl.BlockSpec((_EB, HID), lambda i: (i, 0)),
            pl.BlockSpec((_EB, 16), lambda i: (i, 0)),
        ],
        out_shape=[
            jax.ShapeDtypeStruct((E, HID), jnp.float32),
            jax.ShapeDtypeStruct((E, 16), jnp.float32),
        ],
    )(rg, cg, eah, wd, wa, be, wg, bg, wc, bc)


_NBK = 1264


@functools.partial(jax.jit, static_argnames=("has_h", "do_prep"))
def _tc_node(acc, dsm, cv, dv, xcur, hn_in, hres, wna_h, wna_a, bna, wnb, bnb,
             ln_g, ln_b, wrow, wcol, *, has_h, do_prep):
    def body(*refs):
        i = 0
        accr = refs[i]; i += 1
        dsr = refs[i]; i += 1
        cvr = refs[i]; i += 1
        dvr = refs[i]; i += 1
        xr = refs[i]; i += 1
        if has_h:
            hnr = refs[i]; i += 1
            hrr = refs[i]; i += 1
            wnahr = refs[i]; i += 1
        wnaar = refs[i]; i += 1
        bnar = refs[i]; i += 1
        wnbr = refs[i]; i += 1
        bnbr = refs[i]; i += 1
        if do_prep:
            lngr = refs[i]; i += 1
            lnbr = refs[i]; i += 1
            wrr = refs[i]; i += 1
            wcr = refs[i]; i += 1
        h_out = refs[i]; i += 1
        x_out = refs[i]; i += 1
        if do_prep:
            r_out = refs[i]; i += 1
            c_out = refs[i]; i += 1
            hn_out = refs[i]; i += 1

        na = accr[...] / cvr[...]
        t = _bdot(na, wnaar[...]) + bnar[...]
        if has_h:
            t = t + _bdot(hnr[...], wnahr[...])
        node_out = jax.nn.relu(_bdot(jax.nn.relu(t), wnbr[...]) + bnbr[...])
        h_new = node_out + hrr[...] if has_h else node_out
        lane = lax.broadcasted_iota(jnp.int32, (1, 16), 1)
        x_new = xr[...] + jnp.where(lane < 3, dsr[...], 0.0) / dvr[...]
        h_out[...] = h_new
        x_out[...] = x_new
        if do_prep:
            m = jnp.mean(h_new, axis=1, keepdims=True)
            v = jnp.mean((h_new - m) * (h_new - m), axis=1, keepdims=True)
            hn2 = (h_new - m) / jnp.sqrt(v + EPS) * lngr[...] + lnbr[...]
            zpad = jnp.zeros((x_new.shape[0], HID - 16), jnp.float32)
            r_out[...] = jnp.concatenate([_bdot(hn2, wrr[...]), x_new, zpad], axis=1)
            c_out[...] = jnp.concatenate([_bdot(hn2, wcr[...]), x_new, zpad], axis=1)
            hn_out[...] = hn2

    grid = (NP // _NBK,)
    full = lambda shape: pl.BlockSpec(shape, lambda i: tuple(0 for _ in shape))
    in_specs = [
        pl.BlockSpec((_NBK, HID), lambda i: (i, 0)),
        pl.BlockSpec((_NBK, 16), lambda i: (i, 0)),
        pl.BlockSpec((_NBK, 1), lambda i: (i, 0)),
        pl.BlockSpec((_NBK, 1), lambda i: (i, 0)),
        pl.BlockSpec((_NBK, 16), lambda i: (i, 0)),
    ]
    args = [acc, dsm, cv, dv, xcur]
    if has_h:
        in_specs += [pl.BlockSpec((_NBK, HID), lambda i: (i, 0))] * 2
        in_specs += [full((HID, HID))]
        args += [hn_in, hres, wna_h]
    in_specs += [full((HID, HID)), full((1, HID)), full((HID, HID)),
                 full((1, HID))]
    args += [wna_a, bna, wnb, bnb]
    if do_prep:
        in_specs += [full((1, HID)), full((1, HID)), full((HID, HID)),
                     full((HID, HID))]
        args += [ln_g, ln_b, wrow, wcol]
    out_specs = [
        pl.BlockSpec((_NBK, HID), lambda i: (i, 0)),
        pl.BlockSpec((_NBK, 16), lambda i: (i, 0)),
    ]
    out_shape = [
        jax.ShapeDtypeStruct((NP, HID), jnp.float32),
        jax.ShapeDtypeStruct((NP, 16), jnp.float32),
    ]
    if do_prep:
        out_specs += [pl.BlockSpec((_NBK, 2 * HID), lambda i: (i, 0))] * 2
        out_shape += [jax.ShapeDtypeStruct((NP, 2 * HID), jnp.float32)] * 2
        out_specs += [pl.BlockSpec((_NBK, HID), lambda i: (i, 0))]
        out_shape += [jax.ShapeDtypeStruct((NP, HID), jnp.float32)]
    return pl.pallas_call(
        body, grid=grid, in_specs=in_specs, out_specs=out_specs,
        out_shape=out_shape)(*args)


@jax.jit
def _tc_head(h, fn_g, fn_b, w1, b1, w2, b2, w3p, b3p):
    def body(hr, gr, br, w1r, b1r, w2r, b2r, w3r, b3r, log_ref, emb_ref):
        hv = hr[...]
        m = jnp.mean(hv, axis=1, keepdims=True)
        v = jnp.mean((hv - m) * (hv - m), axis=1, keepdims=True)
        emb = (hv - m) / jnp.sqrt(v + EPS) * gr[...] + br[...]
        z = jax.nn.relu(_bdot(emb, w1r[...]) + b1r[...])
        z = jax.nn.relu(_bdot(z, w2r[...]) + b2r[...])
        log_ref[...] = _bdot(z, w3r[...]) + b3r[...]
        emb_ref[...] = emb

    grid = (NP // _NBK,)
    full = lambda shape: pl.BlockSpec(shape, lambda i: (0, 0))
    return pl.pallas_call(
        body, grid=grid,
        in_specs=[pl.BlockSpec((_NBK, HID), lambda i: (i, 0)),
                  full((1, HID)), full((1, HID)), full((HID, HID)),
                  full((1, HID)), full((HID, HID)), full((1, HID)),
                  full((HID, HID)), full((1, HID))],
        out_specs=[pl.BlockSpec((_NBK, HID), lambda i: (i, 0)),
                   pl.BlockSpec((_NBK, HID), lambda i: (i, 0))],
        out_shape=[jax.ShapeDtypeStruct((NP, HID), jnp.float32),
                   jax.ShapeDtypeStruct((NP, HID), jnp.float32)],
    )(h, fn_g, fn_b, w1, b1, w2, b2, w3p, b3p)


@jax.jit
def _tc_prep(hn_in, ln_g, ln_b, wrow, wcol, x):
    def body(hr, gr, br, wrr, wcr, xr, r_out, c_out, hn_out):
        hv = hr[...]
        m = jnp.mean(hv, axis=1, keepdims=True)
        v = jnp.mean((hv - m) * (hv - m), axis=1, keepdims=True)
        hn2 = (hv - m) / jnp.sqrt(v + EPS) * gr[...] + br[...]
        xv = xr[...]
        zpad = jnp.zeros((xv.shape[0], HID - 16), jnp.float32)
        r_out[...] = jnp.concatenate([_bdot(hn2, wrr[...]), xv, zpad], axis=1)
        c_out[...] = jnp.concatenate([_bdot(hn2, wcr[...]), xv, zpad], axis=1)
        hn_out[...] = hn2

    grid = (NP // _NBK,)
    full = lambda shape: pl.BlockSpec(shape, lambda i: (0, 0))
    return pl.pallas_call(
        body, grid=grid,
        in_specs=[pl.BlockSpec((_NBK, HID), lambda i: (i, 0)),
                  full((1, HID)), full((1, HID)), full((HID, HID)),
                  full((HID, HID)),
                  pl.BlockSpec((_NBK, 16), lambda i: (i, 0))],
        out_specs=[pl.BlockSpec((_NBK, 2 * HID), lambda i: (i, 0)),
                   pl.BlockSpec((_NBK, 2 * HID), lambda i: (i, 0)),
                   pl.BlockSpec((_NBK, HID), lambda i: (i, 0))],
        out_shape=[jax.ShapeDtypeStruct((NP, 2 * HID), jnp.float32),
                   jax.ShapeDtypeStruct((NP, 2 * HID), jnp.float32),
                   jax.ShapeDtypeStruct((NP, HID), jnp.float32)],
    )(hn_in, ln_g, ln_b, wrow, wcol, x)


def kernel(pos_norm, edge_index, edge_angle_hist, center_mask,
           We0, be0, Wc0, bc0, Wg0, bg0, Wna0, bna0, Wnb0, bnb0,
           We, be, Wc, bc, Wg, bg, Wna, bna, Wnb, bnb,
           ln_g, ln_b, fn_g, fn_b, Wc1, bc1, Wc2, bc2, Wc3, bc3):
    f32 = jnp.float32
    x = jnp.pad(pos_norm, ((0, NP - N), (0, 13)))
    row, col = edge_index[0], edge_index[1]
    eah = edge_angle_hist

    perm = jnp.argsort(col)
    rowP = row[perm]
    colP = col[perm]
    perm3 = perm.astype(jnp.int32).reshape(NW, NCHUNK, CH)
    rowP2 = rowP.reshape(NW, EW)
    colP2 = colP.reshape(NW, EW)
    colS3 = colP.reshape(_NBLK, 1, _SB)
    b_of_g, t_of_g, f_of_g = _segsum_schedule(colP)
    cv = jnp.clip(jnp.bincount(col, length=NP).astype(f32), 1.0, None)[:, None]
    dv = jnp.clip(jnp.bincount(row, length=NP).astype(f32), 1.0, None)[:, None]

    eah128 = jnp.pad(eah, ((0, 0), (0, HID - NB)))
    eahP = _sc_gather1(eah128, perm3)[:, :NB]

    r2 = lambda a: a.reshape(1, -1)

    x128 = jnp.pad(x, ((0, 0), (0, HID - 16)))
    rg, cg = _sc_gather(x128, x128, rowP2, colP2, width=HID)
    eo, d = _tc_edge(rg, cg, eahP, r2(We0[0]), We0[1:], r2(be0),
                     r2(Wg0[:, 0]), jnp.broadcast_to(bg0, (1, HID)),
                     r2(Wc0[:, 0]), jnp.broadcast_to(bc0, (1, HID)), hw=0)
    acc_e, ds_e = _tc_segsum(eo, d, colS3, b_of_g, t_of_g, f_of_g)
    h, x = _tc_node(acc_e[:NP], ds_e[:NP], cv, dv, x, None, None, None,
                    Wna0, r2(bna0), Wnb0, r2(bnb0), None, None, None, None,
                    has_h=False, do_prep=False)
    rt, ct, hn = _tc_prep(h, r2(ln_g[0]), r2(ln_b[0]), We[0][:HID],
                          We[0][HID:2 * HID], x)

    for i in range(NL):
        Wei = We[i]
        rg, cg = _sc_gather(rt, ct, rowP2, colP2, width=2 * HID)
        eo, d = _tc_edge(rg, cg, eahP, r2(Wei[2 * HID]), Wei[2 * HID + 1:],
                         r2(be[i]), r2(Wg[i][:, 0]),
                         jnp.broadcast_to(bg[i], (1, HID)),
                         r2(Wc[i][:, 0]), jnp.broadcast_to(bc[i], (1, HID)),
                         hw=HID)
        acc_e, ds_e = _tc_segsum(eo, d, colS3, b_of_g, t_of_g, f_of_g)
        do_prep = i + 1 < NL
        outs = _tc_node(
            acc_e[:NP], ds_e[:NP], cv, dv, x, hn, h, Wna[i][:HID],
            Wna[i][HID:], r2(bna[i]), Wnb[i], r2(bnb[i]),
            r2(ln_g[i + 1]) if do_prep else None,
            r2(ln_b[i + 1]) if do_prep else None,
            We[i + 1][:HID] if do_prep else None,
            We[i + 1][HID:2 * HID] if do_prep else None,
            has_h=True, do_prep=do_prep)
        if do_prep:
            h, x, rt, ct, hn = outs
        else:
            h, x = outs

    w3p = jnp.pad(Wc3, ((0, 0), (0, HID - NC)))
    b3p = jnp.pad(bc3, ((0, HID - NC)))
    logits_p, emb = _tc_head(h, r2(fn_g), r2(fn_b), Wc1, r2(bc1),
                             Wc2, r2(bc2), w3p, r2(b3p))
    return logits_p[:N, :NC], emb[:N]


# speedup vs baseline: 1.1802x; 1.0006x over previous
import functools
import jax
import jax.numpy as jnp
from jax import lax

_PH = jax.lax.Precision.HIGHEST
from jax.experimental import pallas as pl
from jax.experimental.pallas import tpu as pltpu
from jax.experimental.pallas import tpu_sc as plsc

N = 10000
NP = 10112
E = 160000
HID = 128
NB = 16
NL = 3
NC = 10
EPS = 1e-5

NCORE = 2
NSUB = 16
NW = NCORE * NSUB
EW = E // NW
CH = 40
NCHUNK = EW // CH

_mesh = lambda: plsc.VectorSubcoreMesh(core_axis_name="c", subcore_axis_name="s")


def _bdot(a, b):
    return jnp.dot(a.astype(jnp.bfloat16), b.astype(jnp.bfloat16),
                   preferred_element_type=jnp.float32)


def _b16(a):
    return a.astype(jnp.bfloat16).astype(jnp.float32)


CH2 = 64
NFULL = EW // CH2
TAIL = EW - NFULL * CH2


@functools.partial(jax.jit, static_argnames=("width",))
def _sc_gather(rtab, ctab, row2, col2, *, width):
    @functools.partial(
        pl.kernel,
        mesh=_mesh(),
        out_type=[
            jax.ShapeDtypeStruct((E, width), jnp.float32),
            jax.ShapeDtypeStruct((E, width), jnp.float32),
        ],
        scratch_types=[
            pltpu.VMEM((EW,), jnp.int32),
            pltpu.VMEM((EW,), jnp.int32),
            pltpu.VMEM((CH2, width), jnp.float32),
            pltpu.VMEM((CH2, width), jnp.float32),
            pltpu.VMEM((CH2, width), jnp.float32),
            pltpu.VMEM((CH2, width), jnp.float32),
            pltpu.VMEM((CH2, width), jnp.float32),
            pltpu.VMEM((CH2, width), jnp.float32),
            pltpu.SemaphoreType.DMA,
            pltpu.SemaphoreType.DMA,
            pltpu.SemaphoreType.DMA,
            pltpu.SemaphoreType.DMA,
            pltpu.SemaphoreType.DMA,
            pltpu.SemaphoreType.DMA,
        ],
    )
    def gk(rt, ct, r2, c2, rg_out, cg_out, ridx, cidx,
           rb0, rb1, rb2, cb0, cb1, cb2, sr0, sr1, sr2, sc0, sc1, sc2):
        wid = lax.axis_index("s") * NCORE + lax.axis_index("c")
        base = pl.multiple_of(wid * EW, 8)
        pltpu.sync_copy(r2.at[wid], ridx)
        pltpu.sync_copy(c2.at[wid], cidx)
        rb = (rb0, rb1, rb2)
        cb = (cb0, cb1, cb2)
        sr = (sr0, sr1, sr2)
        sc = (sc0, sc1, sc2)
        nch = NFULL + 1
        pend = {}

        def drain(k):
            hr, hc, sz = pend.pop(k)
            hr.wait()
            hc.wait()
            sl = k % 3
            out = pl.ds(base + k * CH2, sz)
            pltpu.sync_copy(rb[sl].at[pl.ds(0, sz)], rg_out.at[out])
            pltpu.sync_copy(cb[sl].at[pl.ds(0, sz)], cg_out.at[out])

        for k in range(nch):
            sz = CH2 if k < NFULL else TAIL
            sl = k % 3
            isl = pl.ds(k * CH2, sz)
            hr = pltpu.async_copy(rt.at[ridx.at[isl]],
                                  rb[sl].at[pl.ds(0, sz)], sr[sl])
            hc = pltpu.async_copy(ct.at[cidx.at[isl]],
                                  cb[sl].at[pl.ds(0, sz)], sc[sl])
            pend[k] = (hr, hc, sz)
            if k >= 2:
                drain(k - 2)
        drain(nch - 2)
        drain(nch - 1)

    return gk(rtab, ctab, row2, col2)


@jax.jit
def _sc_gather1(tab, idx3):
    @functools.partial(
        pl.kernel,
        mesh=_mesh(),
        out_type=jax.ShapeDtypeStruct((E, HID), jnp.float32),
        scratch_types=[
            pltpu.VMEM((NCHUNK, CH), jnp.int32),
            pltpu.VMEM((CH, HID), jnp.float32),
            pltpu.SemaphoreType.DMA,
        ],
    )
    def gk(tb, i3, out, idx, buf, sem):
        wid = lax.axis_index("s") * NCORE + lax.axis_index("c")
        base = wid * EW
        pltpu.sync_copy(i3.at[wid], idx)

        def body(k, carry):
            pltpu.async_copy(tb.at[idx.at[k]], buf, sem).wait()
            off = pl.multiple_of(base + k * CH, 8)
            pltpu.sync_copy(buf, out.at[pl.ds(off, CH)])
            return carry

        lax.fori_loop(0, NCHUNK, body, 0)

    return gk(tab, idx3)


_SB = 1280
_NBLK = E // _SB
_NT = NP // HID
_G = _NBLK + 2 * _NT


@jax.jit
def _tc_segsum(eoS, dS, colS3, b_of_g, t_of_g, f_of_g):
    def body(b_ref, t_ref, f_ref, colr, eor, dr, acc_ref, ds_ref):
        g = pl.program_id(0)
        t = t_ref[g]
        f = f_ref[g]
        colv = colr[0]
        loc = colv - t * HID
        rows = lax.broadcasted_iota(jnp.int32, (HID, 1), 0)
        oh = jnp.where(rows == loc, 1.0, 0.0)
        part = jnp.dot(oh, eor[...], precision=_PH)
        partd = jnp.dot(oh, dr[...], precision=_PH)

        @pl.when(f == 3)
        def _():
            acc_ref[...] = part
            ds_ref[...] = partd

        @pl.when(f == 1)
        def _():
            acc_ref[...] = acc_ref[...] + part
            ds_ref[...] = ds_ref[...] + partd

    grid_spec = pltpu.PrefetchScalarGridSpec(
        num_scalar_prefetch=3,
        grid=(_G,),
        in_specs=[
            pl.BlockSpec((1, 1, _SB), lambda g, b, t, f: (b[g], 0, 0)),
            pl.BlockSpec((_SB, HID), lambda g, b, t, f: (b[g], 0)),
            pl.BlockSpec((_SB, 16), lambda g, b, t, f: (b[g], 0)),
        ],
        out_specs=[
            pl.BlockSpec((HID, HID), lambda g, b, t, f: (t[g], 0)),
            pl.BlockSpec((HID, 16), lambda g, b, t, f: (t[g], 0)),
        ],
    )
    return pl.pallas_call(
        body,
        grid_spec=grid_spec,
        out_shape=[
            jax.ShapeDtypeStruct((NP + HID, HID), jnp.float32),
            jax.ShapeDtypeStruct((NP + HID, 16), jnp.float32),
        ],
    )(b_of_g, t_of_g, f_of_g, colS3, eoS, dS)


def _segsum_schedule(colS):
    bfirst = colS[::_SB] // HID
    blast = colS[_SB - 1::_SB] // HID
    t_ids = jnp.arange(_NT)
    bstart = jnp.searchsorted(blast, t_ids, side="left")
    bend = jnp.searchsorted(bfirst, t_ids, side="right")
    n_t = jnp.maximum(bend - bstart, 1)
    off = jnp.concatenate([jnp.zeros((1,), jnp.int32),
                           jnp.cumsum(n_t).astype(jnp.int32)])
    total = off[-1]
    g_ids = jnp.arange(_G, dtype=jnp.int32)
    t_of_g = jnp.searchsorted(off, g_ids, side="right").astype(jnp.int32) - 1
    t_of_g = jnp.clip(t_of_g, 0, _NT - 1)
    j = g_ids - off[t_of_g]
    empty = (bend - bstart)[t_of_g] <= 0
    b_of_g = jnp.where(empty, 0, bstart[t_of_g] + j).astype(jnp.int32)
    f_of_g = jnp.where(j == 0, 3, 1).astype(jnp.int32)
    pad = g_ids >= total
    t_of_g = jnp.where(pad, _NT, t_of_g).astype(jnp.int32)
    b_of_g = jnp.where(pad, 0, b_of_g)
    f_of_g = jnp.where(pad, 3, f_of_g)
    return b_of_g, t_of_g, f_of_g


_EB = 2000


@functools.partial(jax.jit, static_argnames=("hw",))
def _tc_edge(rg, cg, eah, wd, wa, be, wg, bg, wc, bc, *, hw):
    W = hw + 128

    def body(rgr, cgr, eahr, wdr, war, ber, wgr, bgr, wcr, bcr, eo_ref, d_ref):
        rgv = rgr[...]
        cgv = cgr[...]
        ev = cgv[:, hw:hw + 16] - rgv[:, hw:hw + 16]
        s = jnp.sum(ev * ev, axis=1, keepdims=True)
        dist = jnp.where(s > 0, jnp.sqrt(jnp.where(s > 0, s, 1.0)), 0.0)
        t = _b16(dist) * _b16(wdr[...]) + _bdot(eahr[...], war[...]) + ber[...]
        if hw:
            t = t + rgv[:, :hw] + cgv[:, :hw]
        e1 = jax.nn.relu(t)
        g = jax.nn.sigmoid(jnp.sum(_b16(e1) * _b16(wgr[...]), axis=1,
                                   keepdims=True) + bgr[...][:, :1])
        eo = e1 * g
        w = (jnp.sum(_b16(eo) * _b16(wcr[...]), axis=1, keepdims=True)
             + bcr[...][:, :1])
        lane3 = (lax.broadcasted_iota(jnp.int32, (1, 16), 1) == 3)
        d_ref[...] = ev * w + jnp.where(lane3, 1.0, 0.0)
        eo_ref[...] = eo

    grid = (E // _EB,)
    full = lambda shape: pl.BlockSpec(shape, lambda i: (0, 0))
    return pl.pallas_call(
        body,
        grid=grid,
        in_specs=[
            pl.BlockSpec((_EB, W), lambda i: (i, 0)),
            pl.BlockSpec((_EB, W), lambda i: (i, 0)),
            pl.BlockSpec((_EB, NB), lambda i: (i, 0)),
            full((1, HID)), full((NB, HID)), full((1, HID)),
            full((1, HID)), full((1, HID)), full((1, HID)), full((1, HID)),
        ],
        out_specs=[
            pl.BlockSpec((_EB, HID), lambda i: (i, 0)),
            pl.BlockSpec((_EB, 16), lambda i: (i, 0)),
        ],
        out_shape=[
            jax.ShapeDtypeStruct((E, HID), jnp.float32),
            jax.ShapeDtypeStruct((E, 16), jnp.float32),
        ],
    )(rg, cg, eah, wd, wa, be, wg, bg, wc, bc)


_NBK = 1264


@functools.partial(jax.jit, static_argnames=("has_h", "do_prep"))
def _tc_node(acc, dsm, cv, dv, xcur, hn_in, hres, wna_h, wna_a, bna, wnb, bnb,
             ln_g, ln_b, wrow, wcol, *, has_h, do_prep):
    def body(*refs):
        i = 0
        accr = refs[i]; i += 1
        dsr = refs[i]; i += 1
        cvr = refs[i]; i += 1
        dvr = refs[i]; i += 1
        xr = refs[i]; i += 1
        if has_h:
            hnr = refs[i]; i += 1
            hrr = refs[i]; i += 1
            wnahr = refs[i]; i += 1
        wnaar = refs[i]; i += 1
        bnar = refs[i]; i += 1
        wnbr = refs[i]; i += 1
        bnbr = refs[i]; i += 1
        if do_prep:
            lngr = refs[i]; i += 1
            lnbr = refs[i]; i += 1
            wrr = refs[i]; i += 1
            wcr = refs[i]; i += 1
        h_out = refs[i]; i += 1
        x_out = refs[i]; i += 1
        if do_prep:
            r_out = refs[i]; i += 1
            c_out = refs[i]; i += 1
            hn_out = refs[i]; i += 1

        na = accr[...] / cvr[...]
        t = _bdot(na, wnaar[...]) + bnar[...]
        if has_h:
            t = t + _bdot(hnr[...], wnahr[...])
        node_out = jax.nn.relu(_bdot(jax.nn.relu(t), wnbr[...]) + bnbr[...])
        h_new = node_out + hrr[...] if has_h else node_out
        lane = lax.broadcasted_iota(jnp.int32, (1, 16), 1)
        x_new = xr[...] + jnp.where(lane < 3, dsr[...], 0.0) / dvr[...]
        h_out[...] = h_new
        x_out[...] = x_new
        if do_prep:
            m = jnp.mean(h_new, axis=1, keepdims=True)
            v = jnp.mean((h_new - m) * (h_new - m), axis=1, keepdims=True)
            hn2 = (h_new - m) / jnp.sqrt(v + EPS) * lngr[...] + lnbr[...]
            zpad = jnp.zeros((x_new.shape[0], HID - 16), jnp.float32)
            r_out[...] = jnp.concatenate([_bdot(hn2, wrr[...]), x_new, zpad], axis=1)
            c_out[...] = jnp.concatenate([_bdot(hn2, wcr[...]), x_new, zpad], axis=1)
            hn_out[...] = hn2

    grid = (NP // _NBK,)
    full = lambda shape: pl.BlockSpec(shape, lambda i: tuple(0 for _ in shape))
    in_specs = [
        pl.BlockSpec((_NBK, HID), lambda i: (i, 0)),
        pl.BlockSpec((_NBK, 16), lambda i: (i, 0)),
        pl.BlockSpec((_NBK, 1), lambda i: (i, 0)),
        pl.BlockSpec((_NBK, 1), lambda i: (i, 0)),
        pl.BlockSpec((_NBK, 16), lambda i: (i, 0)),
    ]
    args = [acc, dsm, cv, dv, xcur]
    if has_h:
        in_specs += [pl.BlockSpec((_NBK, HID), lambda i: (i, 0))] * 2
        in_specs += [full((HID, HID))]
        args += [hn_in, hres, wna_h]
    in_specs += [full((HID, HID)), full((1, HID)), full((HID, HID)),
                 full((1, HID))]
    args += [wna_a, bna, wnb, bnb]
    if do_prep:
        in_specs += [full((1, HID)), full((1, HID)), full((HID, HID)),
                     full((HID, HID))]
        args += [ln_g, ln_b, wrow, wcol]
    out_specs = [
        pl.BlockSpec((_NBK, HID), lambda i: (i, 0)),
        pl.BlockSpec((_NBK, 16), lambda i: (i, 0)),
    ]
    out_shape = [
        jax.ShapeDtypeStruct((NP, HID), jnp.float32),
        jax.ShapeDtypeStruct((NP, 16), jnp.float32),
    ]
    if do_prep:
        out_specs += [pl.BlockSpec((_NBK, 2 * HID), lambda i: (i, 0))] * 2
        out_shape += [jax.ShapeDtypeStruct((NP, 2 * HID), jnp.float32)] * 2
        out_specs += [pl.BlockSpec((_NBK, HID), lambda i: (i, 0))]
        out_shape += [jax.ShapeDtypeStruct((NP, HID), jnp.float32)]
    return pl.pallas_call(
        body, grid=grid, in_specs=in_specs, out_specs=out_specs,
        out_shape=out_shape)(*args)


@jax.jit
def _tc_head(h, fn_g, fn_b, w1, b1, w2, b2, w3p, b3p):
    def body(hr, gr, br, w1r, b1r, w2r, b2r, w3r, b3r, log_ref, emb_ref):
        hv = hr[...]
        m = jnp.mean(hv, axis=1, keepdims=True)
        v = jnp.mean((hv - m) * (hv - m), axis=1, keepdims=True)
        emb = (hv - m) / jnp.sqrt(v + EPS) * gr[...] + br[...]
        z = jax.nn.relu(_bdot(emb, w1r[...]) + b1r[...])
        z = jax.nn.relu(_bdot(z, w2r[...]) + b2r[...])
        log_ref[...] = _bdot(z, w3r[...]) + b3r[...]
        emb_ref[...] = emb

    grid = (NP // _NBK,)
    full = lambda shape: pl.BlockSpec(shape, lambda i: (0, 0))
    return pl.pallas_call(
        body, grid=grid,
        in_specs=[pl.BlockSpec((_NBK, HID), lambda i: (i, 0)),
                  full((1, HID)), full((1, HID)), full((HID, HID)),
                  full((1, HID)), full((HID, HID)), full((1, HID)),
                  full((HID, HID)), full((1, HID))],
        out_specs=[pl.BlockSpec((_NBK, HID), lambda i: (i, 0)),
                   pl.BlockSpec((_NBK, HID), lambda i: (i, 0))],
        out_shape=[jax.ShapeDtypeStruct((NP, HID), jnp.float32),
                   jax.ShapeDtypeStruct((NP, HID), jnp.float32)],
    )(h, fn_g, fn_b, w1, b1, w2, b2, w3p, b3p)


@jax.jit
def _tc_prep(hn_in, ln_g, ln_b, wrow, wcol, x):
    def body(hr, gr, br, wrr, wcr, xr, r_out, c_out, hn_out):
        hv = hr[...]
        m = jnp.mean(hv, axis=1, keepdims=True)
        v = jnp.mean((hv - m) * (hv - m), axis=1, keepdims=True)
        hn2 = (hv - m) / jnp.sqrt(v + EPS) * gr[...] + br[...]
        xv = xr[...]
        zpad = jnp.zeros((xv.shape[0], HID - 16), jnp.float32)
        r_out[...] = jnp.concatenate([_bdot(hn2, wrr[...]), xv, zpad], axis=1)
        c_out[...] = jnp.concatenate([_bdot(hn2, wcr[...]), xv, zpad], axis=1)
        hn_out[...] = hn2

    grid = (NP // _NBK,)
    full = lambda shape: pl.BlockSpec(shape, lambda i: (0, 0))
    return pl.pallas_call(
        body, grid=grid,
        in_specs=[pl.BlockSpec((_NBK, HID), lambda i: (i, 0)),
                  full((1, HID)), full((1, HID)), full((HID, HID)),
                  full((HID, HID)),
                  pl.BlockSpec((_NBK, 16), lambda i: (i, 0))],
        out_specs=[pl.BlockSpec((_NBK, 2 * HID), lambda i: (i, 0)),
                   pl.BlockSpec((_NBK, 2 * HID), lambda i: (i, 0)),
                   pl.BlockSpec((_NBK, HID), lambda i: (i, 0))],
        out_shape=[jax.ShapeDtypeStruct((NP, 2 * HID), jnp.float32),
                   jax.ShapeDtypeStruct((NP, 2 * HID), jnp.float32),
                   jax.ShapeDtypeStruct((NP, HID), jnp.float32)],
    )(hn_in, ln_g, ln_b, wrow, wcol, x)


def kernel(pos_norm, edge_index, edge_angle_hist, center_mask,
           We0, be0, Wc0, bc0, Wg0, bg0, Wna0, bna0, Wnb0, bnb0,
           We, be, Wc, bc, Wg, bg, Wna, bna, Wnb, bnb,
           ln_g, ln_b, fn_g, fn_b, Wc1, bc1, Wc2, bc2, Wc3, bc3):
    f32 = jnp.float32
    x = jnp.pad(pos_norm, ((0, NP - N), (0, 13)))
    row, col = edge_index[0], edge_index[1]
    eah = edge_angle_hist

    perm = jnp.argsort(col)
    rowP = row[perm]
    colP = col[perm]
    perm3 = perm.astype(jnp.int32).reshape(NW, NCHUNK, CH)
    rowP2 = rowP.reshape(NW, EW)
    colP2 = colP.reshape(NW, EW)
    colS3 = colP.reshape(_NBLK, 1, _SB)
    b_of_g, t_of_g, f_of_g = _segsum_schedule(colP)
    cv = jnp.clip(jnp.bincount(col, length=NP).astype(f32), 1.0, None)[:, None]
    dv = jnp.clip(jnp.bincount(row, length=NP).astype(f32), 1.0, None)[:, None]

    eah128 = jnp.pad(eah, ((0, 0), (0, HID - NB)))
    eahP = _sc_gather1(eah128, perm3)[:, :NB]

    r2 = lambda a: a.reshape(1, -1)

    x128 = jnp.pad(x, ((0, 0), (0, HID - 16)))
    rg, cg = _sc_gather(x128, x128, rowP2, colP2, width=HID)
    eo, d = _tc_edge(rg, cg, eahP, r2(We0[0]), We0[1:], r2(be0),
                     r2(Wg0[:, 0]), jnp.broadcast_to(bg0, (1, HID)),
                     r2(Wc0[:, 0]), jnp.broadcast_to(bc0, (1, HID)), hw=0)
    acc_e, ds_e = _tc_segsum(eo, d, colS3, b_of_g, t_of_g, f_of_g)
    h, x = _tc_node(acc_e[:NP], ds_e[:NP], cv, dv, x, None, None, None,
                    Wna0, r2(bna0), Wnb0, r2(bnb0), None, None, None, None,
                    has_h=False, do_prep=False)
    rt, ct, hn = _tc_prep(h, r2(ln_g[0]), r2(ln_b[0]), We[0][:HID],
                          We[0][HID:2 * HID], x)

    for i in range(NL):
        Wei = We[i]
        rg, cg = _sc_gather(rt, ct, rowP2, colP2, width=2 * HID)
        eo, d = _tc_edge(rg, cg, eahP, r2(Wei[2 * HID]), Wei[2 * HID + 1:],
                         r2(be[i]), r2(Wg[i][:, 0]),
                         jnp.broadcast_to(bg[i], (1, HID)),
                         r2(Wc[i][:, 0]), jnp.broadcast_to(bc[i], (1, HID)),
                         hw=HID)
        acc_e, ds_e = _tc_segsum(eo, d, colS3, b_of_g, t_of_g, f_of_g)
        do_prep = i + 1 < NL
        outs = _tc_node(
            acc_e[:NP], ds_e[:NP], cv, dv, x, hn, h, Wna[i][:HID],
            Wna[i][HID:], r2(bna[i]), Wnb[i], r2(bnb[i]),
            r2(ln_g[i + 1]) if do_prep else None,
            r2(ln_b[i + 1]) if do_prep else None,
            We[i + 1][:HID] if do_prep else None,
            We[i + 1][HID:2 * HID] if do_prep else None,
            has_h=True, do_prep=do_prep)
        if do_prep:
            h, x, rt, ct, hn = outs
        else:
            h, x = outs

    w3p = jnp.pad(Wc3, ((0, 0), (0, HID - NC)))
    b3p = jnp.pad(bc3, ((0, HID - NC)))
    logits_p, emb = _tc_head(h, r2(fn_g), r2(fn_b), Wc1, r2(bc1),
                             Wc2, r2(bc2), w3p, r2(b3p))
    return logits_p[:N, :NC], emb[:N]
